# Initial kernel scaffold; baseline (speedup 1.0000x reference)
import jax
import jax.numpy as jnp
from jax.experimental import pallas as pl


def kernel(x, text_feat, part_sim, codebook, W1, b1, g1, be1, W2, b2):
    raise NotImplementedError("write your pallas kernel here")


# speedup vs baseline: 3.0427x; 3.0427x over previous
import jax
import jax.numpy as jnp
from jax.experimental import pallas as pl
from jax.experimental.pallas import tpu as pltpu

NB = 1024
CD = 256
TXT = 512
HID = 512
N_BATCH = 32
T_LEN = 512
KB = 128


def _gate_kernel(tf_ref, ps_ref, cb_ref, w1a_ref, w1b_ref, b1_ref, g1_ref,
                 be1_ref, w2_ref, b2_ref, tmp_ref, mix_ref):
    cb = cb_ref[...]
    a = jnp.dot(tf_ref[...], w1a_ref[...],
                preferred_element_type=jnp.float32)
    c = jnp.dot(cb, w1b_ref[...],
                preferred_element_type=jnp.float32)
    h = c[:, None, :] + a[None, :, :] + b1_ref[...][None, :, :]
    mu = jnp.mean(h, axis=-1, keepdims=True)
    var = jnp.mean((h - mu) ** 2, axis=-1, keepdims=True)
    h = (h - mu) * jax.lax.rsqrt(var + 1e-5)
    h = h * g1_ref[...][None, :, :] + be1_ref[...][None, :, :]
    h = jnp.where(h >= 0, h, 0.01 * h)
    h2 = h.reshape(KB * N_BATCH, HID)
    g = jnp.dot(h2, w2_ref[...], preferred_element_type=jnp.float32)
    g = jax.nn.sigmoid(g + b2_ref[...])
    g = g.reshape(KB, N_BATCH, CD)
    ps = ps_ref[...].reshape(1, N_BATCH, 1)
    dmean = jnp.sum(g * ps, axis=1) * (1.0 / N_BATCH)
    tmp_ref[...] = cb + 0.3 * dmean
    mix_ref[...] = cb + 0.09 * dmean


def _main_kernel(x_ref, cb_ref, tmp_ref, mix_ref, xd_ref, commit_ref, perp_ref,
                 counts_acc, loss_acc):
    n = pl.program_id(0)

    @pl.when(n == 0)
    def _init():
        counts_acc[...] = jnp.zeros_like(counts_acc)
        loss_acc[0] = 0.0
        loss_acc[1] = 0.0

    xn = x_ref[0]
    cb = cb_ref[...]
    tmp = tmp_ref[...]
    mix = mix_ref[...]

    mm0 = jnp.dot(cb, xn, preferred_element_type=jnp.float32)
    mm1 = jnp.dot(tmp, xn, preferred_element_type=jnp.float32)

    x_sq = jnp.sum(xn * xn, axis=0, keepdims=True)
    cb_sq = jnp.sum(cb * cb, axis=1, keepdims=True)
    tmp_sq = jnp.sum(tmp * tmp, axis=1, keepdims=True)
    mix_sq = jnp.sum(mix * mix, axis=1, keepdims=True)

    iota = jax.lax.broadcasted_iota(jnp.int32, (NB, T_LEN), 0)

    d0 = x_sq + cb_sq - 2.0 * mm0
    min0 = jnp.min(d0, axis=0, keepdims=True)
    idx0 = jnp.min(jnp.where(d0 == min0, iota, NB), axis=0, keepdims=True)
    oh0 = (iota == idx0).astype(jnp.float32)
    counts_acc[...] += jnp.sum(oh0, axis=1, keepdims=True)
    loss_acc[0] += jnp.sum(min0)

    d1 = jnp.maximum(x_sq + tmp_sq - 2.0 * mm1, 0.0)
    min1 = jnp.min(d1, axis=0, keepdims=True)
    idx1 = jnp.min(jnp.where(d1 == min1, iota, NB), axis=0, keepdims=True)
    oh1 = (iota == idx1).astype(jnp.float32)

    xd_ref[0] = jax.lax.dot_general(mix, oh1, (((0,), (0,)), ((), ())),
                                    preferred_element_type=jnp.float32)

    mmix = 0.7 * mm0 + 0.3 * mm1
    val = jnp.sum(oh1 * mmix, axis=0, keepdims=True)
    s_at = jnp.sum(oh1 * mix_sq, axis=0, keepdims=True)
    loss_acc[1] += jnp.sum(x_sq - 2.0 * val + s_at)

    @pl.when(n == N_BATCH - 1)
    def _final():
        total = jnp.float32(N_BATCH * T_LEN)
        p = counts_acc[...] * (1.0 / total)
        plogp = jnp.where(p > 0.0, p * jnp.log(jnp.maximum(p, 1e-30)), 0.0)
        perp_ref[...] = jnp.exp(-jnp.sum(plogp, axis=0, keepdims=True))
        denom = 1.0 / (total * CD)
        commit = (0.7 * loss_acc[0] + 0.3 * loss_acc[1]) * denom
        commit_ref[...] = commit * jnp.ones((1, 1), jnp.float32)


def kernel(x, text_feat, part_sim, codebook, W1, b1, g1, be1, W2, b2):
    f32 = jnp.float32
    W1a = W1[:TXT]
    W1b = W1[TXT:]
    b1r = b1.reshape(1, HID)
    g1r = g1.reshape(1, HID)
    be1r = be1.reshape(1, HID)
    b2r = b2.reshape(1, CD)
    psr = part_sim.reshape(1, N_BATCH)

    n_kb = NB // KB
    tmp_cb, mix_cb = pl.pallas_call(
        _gate_kernel,
        grid=(n_kb,),
        in_specs=[
            pl.BlockSpec((N_BATCH, TXT), lambda i: (0, 0)),
            pl.BlockSpec((1, N_BATCH), lambda i: (0, 0)),
            pl.BlockSpec((KB, CD), lambda i: (i, 0)),
            pl.BlockSpec((TXT, HID), lambda i: (0, 0)),
            pl.BlockSpec((CD, HID), lambda i: (0, 0)),
            pl.BlockSpec((1, HID), lambda i: (0, 0)),
            pl.BlockSpec((1, HID), lambda i: (0, 0)),
            pl.BlockSpec((1, HID), lambda i: (0, 0)),
            pl.BlockSpec((HID, CD), lambda i: (0, 0)),
            pl.BlockSpec((1, CD), lambda i: (0, 0)),
        ],
        out_specs=[
            pl.BlockSpec((KB, CD), lambda i: (i, 0)),
            pl.BlockSpec((KB, CD), lambda i: (i, 0)),
        ],
        out_shape=[
            jax.ShapeDtypeStruct((NB, CD), f32),
            jax.ShapeDtypeStruct((NB, CD), f32),
        ],
    )(text_feat, psr, codebook, W1a, W1b, b1r, g1r, be1r, W2, b2r)

    x_d, commit, perp = pl.pallas_call(
        _main_kernel,
        grid=(N_BATCH,),
        in_specs=[
            pl.BlockSpec((1, CD, T_LEN), lambda n: (n, 0, 0)),
            pl.BlockSpec((NB, CD), lambda n: (0, 0)),
            pl.BlockSpec((NB, CD), lambda n: (0, 0)),
            pl.BlockSpec((NB, CD), lambda n: (0, 0)),
        ],
        out_specs=[
            pl.BlockSpec((1, CD, T_LEN), lambda n: (n, 0, 0)),
            pl.BlockSpec((1, 1), lambda n: (0, 0)),
            pl.BlockSpec((1, 1), lambda n: (0, 0)),
        ],
        out_shape=[
            jax.ShapeDtypeStruct((N_BATCH, CD, T_LEN), f32),
            jax.ShapeDtypeStruct((1, 1), f32),
            jax.ShapeDtypeStruct((1, 1), f32),
        ],
        scratch_shapes=[
            pltpu.VMEM((NB, 1), f32),
            pltpu.SMEM((2,), f32),
        ],
    )(x, codebook, tmp_cb, mix_cb)

    return (x_d, commit[0, 0], perp[0, 0])


# speedup vs baseline: 3.3460x; 1.0997x over previous
import jax
import jax.numpy as jnp
from jax.experimental import pallas as pl
from jax.experimental.pallas import tpu as pltpu

NB = 1024
CD = 256
TXT = 512
HID = 512
N_BATCH = 32
T_LEN = 512
KB = 128


def _gate_kernel(tf_ref, ps_ref, cb_ref, w1a_ref, w1b_ref, b1_ref, g1_ref,
                 be1_ref, w2_ref, b2_ref,
                 tmps_ref, mix_ref, cbsq_ref, tmpsq_ref):
    cb = cb_ref[...]
    a = jnp.dot(tf_ref[...], w1a_ref[...],
                preferred_element_type=jnp.float32)
    c = jnp.dot(cb, w1b_ref[...],
                preferred_element_type=jnp.float32)
    b1 = b1_ref[...]
    cp = c + b1
    mu_a = jnp.mean(a, axis=1, keepdims=True)
    A = a - mu_a
    vA = jnp.mean(A * A, axis=1, keepdims=True)
    mu_c = jnp.mean(cp, axis=1, keepdims=True)
    C = cp - mu_c
    vC = jnp.mean(C * C, axis=1, keepdims=True)
    cross = jax.lax.dot_general(C, A, (((1,), (1,)), ((), ())),
                                precision=jax.lax.Precision.HIGHEST,
                                preferred_element_type=jnp.float32)
    var = vC + vA.reshape(1, N_BATCH) + cross * (2.0 / HID)
    inv = jax.lax.rsqrt(var + 1e-5)
    mu = mu_c + mu_a.reshape(1, N_BATCH)
    t = (c[:, None, :] + a[None, :, :]) + b1[None, :, :]
    t = (t - mu[:, :, None]) * inv[:, :, None]
    t = t * g1_ref[...][None, :, :] + be1_ref[...][None, :, :]
    t = jnp.where(t >= 0, t, 0.01 * t)
    g = jnp.dot(t.reshape(KB * N_BATCH, HID), w2_ref[...],
                preferred_element_type=jnp.float32)
    g = jax.nn.sigmoid(g + b2_ref[...])
    g3 = g.reshape(KB, N_BATCH, CD) * ps_ref[...].reshape(1, N_BATCH, 1)
    dmean = jnp.sum(g3, axis=1) * (1.0 / N_BATCH)
    tmp = 0.7 * cb + 0.3 * (cb + dmean)
    tmps_ref[...] = tmp
    mix_ref[...] = 0.7 * cb + 0.3 * tmp
    cbsq_ref[...] = jnp.sum(cb * cb, axis=1, keepdims=True)
    tmpsq_ref[...] = jnp.sum(tmp * tmp, axis=1, keepdims=True)


def _main_kernel(x_ref, cb_ref, tmps_ref, mix_ref, cbsq_ref, tmpsq_ref,
                 xd_ref, commit_ref, perp_ref, counts_acc, loss_acc):
    n = pl.program_id(0)

    @pl.when(n == 0)
    def _init():
        counts_acc[...] = jnp.zeros_like(counts_acc)
        loss_acc[0] = 0.0
        loss_acc[1] = 0.0

    xn = x_ref[0]
    x_sq = jnp.sum(xn * xn, axis=0, keepdims=True)
    xn2 = -2.0 * xn
    mm0 = jnp.dot(cb_ref[...], xn2,
                  preferred_element_type=jnp.float32)
    d0 = (x_sq + mm0) + cbsq_ref[...]
    min0 = jnp.min(d0, axis=0, keepdims=True)
    eq0 = jnp.where(d0 == min0, 1.0, 0.0)
    ones_t = jnp.ones((T_LEN, 1), jnp.float32)
    counts_acc[...] += jnp.dot(eq0, ones_t, preferred_element_type=jnp.float32)
    loss_acc[0] += jnp.sum(min0)

    mm1 = jnp.dot(tmps_ref[...], xn2,
                  preferred_element_type=jnp.float32)
    d1 = jnp.sqrt(jnp.maximum((x_sq + mm1) + tmpsq_ref[...], 0.0))
    min1 = jnp.min(d1, axis=0, keepdims=True)
    iota = jax.lax.broadcasted_iota(jnp.int32, (NB, T_LEN), 0).astype(jnp.float32)
    idx1 = jnp.min(jnp.where(d1 == min1, iota, jnp.float32(NB)),
                   axis=0, keepdims=True)
    oh1 = jnp.where(iota == idx1, 1.0, 0.0)
    out = jax.lax.dot_general(mix_ref[...], oh1, (((0,), (0,)), ((), ())),
                              preferred_element_type=jnp.float32)
    xd_ref[0] = out
    diff = xn - out
    loss_acc[1] += jnp.sum(diff * diff)

    @pl.when(n == N_BATCH - 1)
    def _final():
        cnt = counts_acc[...]
        p = cnt * (1.0 / jnp.sum(cnt))
        plogp = jnp.where(p > 0.0, p * jnp.log(jnp.maximum(p, 1e-30)), 0.0)
        perp_ref[...] = jnp.exp(-jnp.sum(plogp, axis=0, keepdims=True))
        denom = 1.0 / (jnp.float32(N_BATCH * T_LEN) * CD)
        commit = (0.7 * loss_acc[0] + 0.3 * loss_acc[1]) * denom
        commit_ref[...] = commit * jnp.ones((1, 1), jnp.float32)


def kernel(x, text_feat, part_sim, codebook, W1, b1, g1, be1, W2, b2):
    f32 = jnp.float32
    W1a = W1[:TXT]
    W1b = W1[TXT:]
    b1r = b1.reshape(1, HID)
    g1r = g1.reshape(1, HID)
    be1r = be1.reshape(1, HID)
    b2r = b2.reshape(1, CD)
    psr = part_sim.reshape(1, N_BATCH)

    n_kb = NB // KB
    full = lambda i: (0, 0)
    blk = lambda i: (i, 0)
    tmps, mix_cb, cbsq, tmpsq = pl.pallas_call(
        _gate_kernel,
        grid=(n_kb,),
        in_specs=[
            pl.BlockSpec((N_BATCH, TXT), full),
            pl.BlockSpec((1, N_BATCH), full),
            pl.BlockSpec((KB, CD), blk),
            pl.BlockSpec((TXT, HID), full),
            pl.BlockSpec((CD, HID), full),
            pl.BlockSpec((1, HID), full),
            pl.BlockSpec((1, HID), full),
            pl.BlockSpec((1, HID), full),
            pl.BlockSpec((HID, CD), full),
            pl.BlockSpec((1, CD), full),
        ],
        out_specs=[
            pl.BlockSpec((KB, CD), blk),
            pl.BlockSpec((KB, CD), blk),
            pl.BlockSpec((KB, 1), blk),
            pl.BlockSpec((KB, 1), blk),
        ],
        out_shape=[
            jax.ShapeDtypeStruct((NB, CD), f32),
            jax.ShapeDtypeStruct((NB, CD), f32),
            jax.ShapeDtypeStruct((NB, 1), f32),
            jax.ShapeDtypeStruct((NB, 1), f32),
        ],
    )(text_feat, psr, codebook, W1a, W1b, b1r, g1r, be1r, W2, b2r)

    x_d, commit, perp = pl.pallas_call(
        _main_kernel,
        grid=(N_BATCH,),
        in_specs=[
            pl.BlockSpec((1, CD, T_LEN), lambda n: (n, 0, 0)),
            pl.BlockSpec((NB, CD), full),
            pl.BlockSpec((NB, CD), full),
            pl.BlockSpec((NB, CD), full),
            pl.BlockSpec((NB, 1), full),
            pl.BlockSpec((NB, 1), full),
        ],
        out_specs=[
            pl.BlockSpec((1, CD, T_LEN), lambda n: (n, 0, 0)),
            pl.BlockSpec((1, 1), full),
            pl.BlockSpec((1, 1), full),
        ],
        out_shape=[
            jax.ShapeDtypeStruct((N_BATCH, CD, T_LEN), f32),
            jax.ShapeDtypeStruct((1, 1), f32),
            jax.ShapeDtypeStruct((1, 1), f32),
        ],
        scratch_shapes=[
            pltpu.VMEM((NB, 1), f32),
            pltpu.SMEM((2,), f32),
        ],
    )(x, codebook, tmps, mix_cb, cbsq, tmpsq)

    return (x_d, commit[0, 0], perp[0, 0])


# speedup vs baseline: 3.4304x; 1.0252x over previous
import jax
import jax.numpy as jnp
from jax.experimental import pallas as pl
from jax.experimental.pallas import tpu as pltpu

NB = 1024
CD = 256
TXT = 512
HID = 512
N_BATCH = 32
T_LEN = 512
KB = 128


def _gate_kernel(tf_ref, ps_ref, cb_ref, w1a_ref, w1b_ref, b1_ref, g1_ref,
                 be1_ref, w2_ref, b2_ref,
                 tmps_ref, mix_ref, cbsq_ref, tmpsq_ref):
    cb = cb_ref[...]
    a = jnp.dot(tf_ref[...], w1a_ref[...],
                preferred_element_type=jnp.float32)
    c = jnp.dot(cb, w1b_ref[...],
                preferred_element_type=jnp.float32)
    b1 = b1_ref[...]
    cp = c + b1
    mu_a = jnp.mean(a, axis=1, keepdims=True)
    A = a - mu_a
    vA = jnp.mean(A * A, axis=1, keepdims=True)
    mu_c = jnp.mean(cp, axis=1, keepdims=True)
    C = cp - mu_c
    vC = jnp.mean(C * C, axis=1, keepdims=True)
    cross = jax.lax.dot_general(C, A, (((1,), (1,)), ((), ())),
                                precision=jax.lax.Precision.HIGHEST,
                                preferred_element_type=jnp.float32)
    var = vC + vA.reshape(1, N_BATCH) + cross * (2.0 / HID)
    inv = jax.lax.rsqrt(var + 1e-5)
    mu = mu_c + mu_a.reshape(1, N_BATCH)
    ab1 = a + b1
    t = c[:, None, :] + ab1[None, :, :]
    t = (t - mu[:, :, None]) * inv[:, :, None]
    t = t * g1_ref[...][None, :, :] + be1_ref[...][None, :, :]
    t = jnp.maximum(t, 0.01 * t)
    g = jnp.dot(t.reshape(KB * N_BATCH, HID), w2_ref[...],
                preferred_element_type=jnp.float32)
    g = jax.nn.sigmoid(g + b2_ref[...])
    g3 = g.reshape(KB, N_BATCH, CD) * ps_ref[...].reshape(1, N_BATCH, 1)
    dmean = jnp.sum(g3, axis=1) * (1.0 / N_BATCH)
    tmp = 0.7 * cb + 0.3 * (cb + dmean)
    tmps_ref[...] = tmp
    mix_ref[...] = 0.7 * cb + 0.3 * tmp
    cbsq_ref[...] = jnp.sum(cb * cb, axis=1, keepdims=True)
    tmpsq_ref[...] = jnp.sum(tmp * tmp, axis=1, keepdims=True)


def _main_kernel(x_ref, cb_ref, tmps_ref, mix_ref, cbsq_ref, tmpsq_ref,
                 xd_ref, commit_ref, perp_ref, counts_acc, loss_acc):
    n = pl.program_id(0)

    @pl.when(n == 0)
    def _init():
        counts_acc[...] = jnp.zeros_like(counts_acc)
        loss_acc[0] = 0.0
        loss_acc[1] = 0.0

    xn = x_ref[0]
    x_sq = jnp.sum(xn * xn, axis=0, keepdims=True)
    xn2 = -2.0 * xn
    mm0 = jnp.dot(cb_ref[...], xn2,
                  preferred_element_type=jnp.float32)
    d0 = (x_sq + mm0) + cbsq_ref[...]
    min0 = jnp.min(d0, axis=0, keepdims=True)
    eq0 = jnp.where(d0 == min0, 1.0, 0.0)
    ones_t = jnp.ones((T_LEN, 1), jnp.float32)
    counts_acc[...] += jnp.dot(eq0, ones_t, preferred_element_type=jnp.float32)
    loss_acc[0] += jnp.sum(min0)

    mm1 = jnp.dot(tmps_ref[...], xn2,
                  preferred_element_type=jnp.float32)
    d1 = jnp.sqrt(jnp.maximum((x_sq + mm1) + tmpsq_ref[...], 0.0))
    min1 = jnp.min(d1, axis=0, keepdims=True)
    iota = jax.lax.broadcasted_iota(jnp.int32, (NB, T_LEN), 0).astype(jnp.float32)
    idx1 = jnp.min(jnp.where(d1 == min1, iota, jnp.float32(NB)),
                   axis=0, keepdims=True)
    oh1 = jnp.where(iota == idx1, 1.0, 0.0)
    out = jax.lax.dot_general(mix_ref[...], oh1, (((0,), (0,)), ((), ())),
                              preferred_element_type=jnp.float32)
    xd_ref[0] = out
    diff = xn - out
    loss_acc[1] += jnp.sum(diff * diff)

    @pl.when(n == N_BATCH - 1)
    def _final():
        cnt = counts_acc[...]
        p = cnt * (1.0 / jnp.sum(cnt))
        plogp = jnp.where(p > 0.0, p * jnp.log(jnp.maximum(p, 1e-30)), 0.0)
        perp_ref[...] = jnp.exp(-jnp.sum(plogp, axis=0, keepdims=True))
        denom = 1.0 / (jnp.float32(N_BATCH * T_LEN) * CD)
        commit = (0.7 * loss_acc[0] + 0.3 * loss_acc[1]) * denom
        commit_ref[...] = commit * jnp.ones((1, 1), jnp.float32)


def kernel(x, text_feat, part_sim, codebook, W1, b1, g1, be1, W2, b2):
    f32 = jnp.float32
    W1a = W1[:TXT]
    W1b = W1[TXT:]
    b1r = b1.reshape(1, HID)
    g1r = g1.reshape(1, HID)
    be1r = be1.reshape(1, HID)
    b2r = b2.reshape(1, CD)
    psr = part_sim.reshape(1, N_BATCH)

    n_kb = NB // KB
    full = lambda i: (0, 0)
    blk = lambda i: (i, 0)
    tmps, mix_cb, cbsq, tmpsq = pl.pallas_call(
        _gate_kernel,
        grid=(n_kb,),
        in_specs=[
            pl.BlockSpec((N_BATCH, TXT), full),
            pl.BlockSpec((1, N_BATCH), full),
            pl.BlockSpec((KB, CD), blk),
            pl.BlockSpec((TXT, HID), full),
            pl.BlockSpec((CD, HID), full),
            pl.BlockSpec((1, HID), full),
            pl.BlockSpec((1, HID), full),
            pl.BlockSpec((1, HID), full),
            pl.BlockSpec((HID, CD), full),
            pl.BlockSpec((1, CD), full),
        ],
        out_specs=[
            pl.BlockSpec((KB, CD), blk),
            pl.BlockSpec((KB, CD), blk),
            pl.BlockSpec((KB, 1), blk),
            pl.BlockSpec((KB, 1), blk),
        ],
        out_shape=[
            jax.ShapeDtypeStruct((NB, CD), f32),
            jax.ShapeDtypeStruct((NB, CD), f32),
            jax.ShapeDtypeStruct((NB, 1), f32),
            jax.ShapeDtypeStruct((NB, 1), f32),
        ],
    )(text_feat, psr, codebook, W1a, W1b, b1r, g1r, be1r, W2, b2r)

    x_d, commit, perp = pl.pallas_call(
        _main_kernel,
        grid=(N_BATCH,),
        in_specs=[
            pl.BlockSpec((1, CD, T_LEN), lambda n: (n, 0, 0)),
            pl.BlockSpec((NB, CD), full),
            pl.BlockSpec((NB, CD), full),
            pl.BlockSpec((NB, CD), full),
            pl.BlockSpec((NB, 1), full),
            pl.BlockSpec((NB, 1), full),
        ],
        out_specs=[
            pl.BlockSpec((1, CD, T_LEN), lambda n: (n, 0, 0)),
            pl.BlockSpec((1, 1), full),
            pl.BlockSpec((1, 1), full),
        ],
        out_shape=[
            jax.ShapeDtypeStruct((N_BATCH, CD, T_LEN), f32),
            jax.ShapeDtypeStruct((1, 1), f32),
            jax.ShapeDtypeStruct((1, 1), f32),
        ],
        scratch_shapes=[
            pltpu.VMEM((NB, 1), f32),
            pltpu.SMEM((2,), f32),
        ],
    )(x, codebook, tmps, mix_cb, cbsq, tmpsq)

    return (x_d, commit[0, 0], perp[0, 0])


# speedup vs baseline: 3.4649x; 1.0101x over previous
import jax
import jax.numpy as jnp
from jax.experimental import pallas as pl
from jax.experimental.pallas import tpu as pltpu

NB = 1024
CD = 256
TXT = 512
HID = 512
N_BATCH = 32
T_LEN = 512
KB = 256


def _gate_kernel(tf_ref, ps_ref, cb_ref, w1a_ref, w1b_ref, b1_ref, g1_ref,
                 be1_ref, w2_ref, b2_ref,
                 tmps_ref, mix_ref, cbsq_ref, tmpsq_ref):
    cb = cb_ref[...]
    a = jnp.dot(tf_ref[...], w1a_ref[...],
                preferred_element_type=jnp.float32)
    c = jnp.dot(cb, w1b_ref[...],
                preferred_element_type=jnp.float32)
    b1 = b1_ref[...]
    cp = c + b1
    mu_a = jnp.mean(a, axis=1, keepdims=True)
    A = a - mu_a
    vA = jnp.mean(A * A, axis=1, keepdims=True)
    mu_c = jnp.mean(cp, axis=1, keepdims=True)
    C = cp - mu_c
    vC = jnp.mean(C * C, axis=1, keepdims=True)
    cross = jax.lax.dot_general(C, A, (((1,), (1,)), ((), ())),
                                precision=jax.lax.Precision.HIGHEST,
                                preferred_element_type=jnp.float32)
    var = vC + vA.reshape(1, N_BATCH) + cross * (2.0 / HID)
    inv = jax.lax.rsqrt(var + 1e-5)
    mu = mu_c + mu_a.reshape(1, N_BATCH)
    ab1 = a + b1
    t = c[:, None, :] + ab1[None, :, :]
    t = (t - mu[:, :, None]) * inv[:, :, None]
    t = t * g1_ref[...][None, :, :] + be1_ref[...][None, :, :]
    t = jnp.maximum(t, 0.01 * t)
    g = jnp.dot(t.reshape(KB * N_BATCH, HID), w2_ref[...],
                preferred_element_type=jnp.float32)
    g = jax.nn.sigmoid(g + b2_ref[...])
    g3 = g.reshape(KB, N_BATCH, CD) * ps_ref[...].reshape(1, N_BATCH, 1)
    dmean = jnp.sum(g3, axis=1) * (1.0 / N_BATCH)
    tmp = 0.7 * cb + 0.3 * (cb + dmean)
    tmps_ref[...] = tmp
    mix_ref[...] = 0.7 * cb + 0.3 * tmp
    cbsq_ref[...] = jnp.sum(cb * cb, axis=1, keepdims=True)
    tmpsq_ref[...] = jnp.sum(tmp * tmp, axis=1, keepdims=True)


def _main_kernel(x_ref, cb_ref, tmps_ref, mix_ref, cbsq_ref, tmpsq_ref,
                 xd_ref, commit_ref, perp_ref, counts_acc, loss_acc):
    n = pl.program_id(0)

    @pl.when(n == 0)
    def _init():
        counts_acc[...] = jnp.zeros_like(counts_acc)
        loss_acc[0] = 0.0
        loss_acc[1] = 0.0

    xn = x_ref[0]
    x_sq = jnp.sum(xn * xn, axis=0, keepdims=True)
    xn2 = -2.0 * xn
    mm0 = jnp.dot(cb_ref[...], xn2,
                  preferred_element_type=jnp.float32)
    d0 = (x_sq + mm0) + cbsq_ref[...]
    min0 = jnp.min(d0, axis=0, keepdims=True)
    eq0 = jnp.where(d0 == min0, 1.0, 0.0)
    ones_t = jnp.ones((T_LEN, 1), jnp.float32)
    counts_acc[...] += jnp.dot(eq0, ones_t, preferred_element_type=jnp.float32)
    loss_acc[0] += jnp.sum(min0)

    mm1 = jnp.dot(tmps_ref[...], xn2,
                  preferred_element_type=jnp.float32)
    d1 = jnp.sqrt(jnp.maximum((x_sq + mm1) + tmpsq_ref[...], 0.0))
    min1 = jnp.min(d1, axis=0, keepdims=True)
    iota = jax.lax.broadcasted_iota(jnp.int32, (NB, T_LEN), 0).astype(jnp.float32)
    idx1 = jnp.min(jnp.where(d1 == min1, iota, jnp.float32(NB)),
                   axis=0, keepdims=True)
    oh1 = jnp.where(iota == idx1, 1.0, 0.0)
    out = jax.lax.dot_general(mix_ref[...], oh1, (((0,), (0,)), ((), ())),
                              preferred_element_type=jnp.float32)
    xd_ref[0] = out
    diff = xn - out
    loss_acc[1] += jnp.sum(diff * diff)

    @pl.when(n == N_BATCH - 1)
    def _final():
        cnt = counts_acc[...]
        p = cnt * (1.0 / jnp.sum(cnt))
        plogp = jnp.where(p > 0.0, p * jnp.log(jnp.maximum(p, 1e-30)), 0.0)
        perp_ref[...] = jnp.exp(-jnp.sum(plogp, axis=0, keepdims=True))
        denom = 1.0 / (jnp.float32(N_BATCH * T_LEN) * CD)
        commit = (0.7 * loss_acc[0] + 0.3 * loss_acc[1]) * denom
        commit_ref[...] = commit * jnp.ones((1, 1), jnp.float32)


def kernel(x, text_feat, part_sim, codebook, W1, b1, g1, be1, W2, b2):
    f32 = jnp.float32
    W1a = W1[:TXT]
    W1b = W1[TXT:]
    b1r = b1.reshape(1, HID)
    g1r = g1.reshape(1, HID)
    be1r = be1.reshape(1, HID)
    b2r = b2.reshape(1, CD)
    psr = part_sim.reshape(1, N_BATCH)

    n_kb = NB // KB
    full = lambda i: (0, 0)
    blk = lambda i: (i, 0)
    tmps, mix_cb, cbsq, tmpsq = pl.pallas_call(
        _gate_kernel,
        grid=(n_kb,),
        in_specs=[
            pl.BlockSpec((N_BATCH, TXT), full),
            pl.BlockSpec((1, N_BATCH), full),
            pl.BlockSpec((KB, CD), blk),
            pl.BlockSpec((TXT, HID), full),
            pl.BlockSpec((CD, HID), full),
            pl.BlockSpec((1, HID), full),
            pl.BlockSpec((1, HID), full),
            pl.BlockSpec((1, HID), full),
            pl.BlockSpec((HID, CD), full),
            pl.BlockSpec((1, CD), full),
        ],
        out_specs=[
            pl.BlockSpec((KB, CD), blk),
            pl.BlockSpec((KB, CD), blk),
            pl.BlockSpec((KB, 1), blk),
            pl.BlockSpec((KB, 1), blk),
        ],
        out_shape=[
            jax.ShapeDtypeStruct((NB, CD), f32),
            jax.ShapeDtypeStruct((NB, CD), f32),
            jax.ShapeDtypeStruct((NB, 1), f32),
            jax.ShapeDtypeStruct((NB, 1), f32),
        ],
    )(text_feat, psr, codebook, W1a, W1b, b1r, g1r, be1r, W2, b2r)

    x_d, commit, perp = pl.pallas_call(
        _main_kernel,
        grid=(N_BATCH,),
        in_specs=[
            pl.BlockSpec((1, CD, T_LEN), lambda n: (n, 0, 0)),
            pl.BlockSpec((NB, CD), full),
            pl.BlockSpec((NB, CD), full),
            pl.BlockSpec((NB, CD), full),
            pl.BlockSpec((NB, 1), full),
            pl.BlockSpec((NB, 1), full),
        ],
        out_specs=[
            pl.BlockSpec((1, CD, T_LEN), lambda n: (n, 0, 0)),
            pl.BlockSpec((1, 1), full),
            pl.BlockSpec((1, 1), full),
        ],
        out_shape=[
            jax.ShapeDtypeStruct((N_BATCH, CD, T_LEN), f32),
            jax.ShapeDtypeStruct((1, 1), f32),
            jax.ShapeDtypeStruct((1, 1), f32),
        ],
        scratch_shapes=[
            pltpu.VMEM((NB, 1), f32),
            pltpu.SMEM((2,), f32),
        ],
    )(x, codebook, tmps, mix_cb, cbsq, tmpsq)

    return (x_d, commit[0, 0], perp[0, 0])


# speedup vs baseline: 3.4840x; 1.0055x over previous
import jax
import jax.numpy as jnp
from jax.experimental import pallas as pl
from jax.experimental.pallas import tpu as pltpu

NB = 1024
CD = 256
TXT = 512
HID = 512
N_BATCH = 32
T_LEN = 512
KB = 128
NGATE = NB // KB


def _fused_kernel(tf_ref, ps_ref, cb_ref, w1a_ref, w1b_ref, b1_ref, g1_ref,
                  be1_ref, w2_ref, b2_ref, x_ref,
                  xd_ref, commit_ref, perp_ref,
                  tmp_s, mix_s, cbsq_s, tmpsq_s, counts_acc, loss_acc):
    i = pl.program_id(0)

    @pl.when(i < NGATE)
    def _gate():
        cb = cb_ref[pl.ds(i * KB, KB), :]
        a = jnp.dot(tf_ref[...], w1a_ref[...],
                    preferred_element_type=jnp.float32)
        c = jnp.dot(cb, w1b_ref[...],
                    preferred_element_type=jnp.float32)
        b1 = b1_ref[...]
        cp = c + b1
        mu_a = jnp.mean(a, axis=1, keepdims=True)
        A = a - mu_a
        vA = jnp.mean(A * A, axis=1, keepdims=True)
        mu_c = jnp.mean(cp, axis=1, keepdims=True)
        C = cp - mu_c
        vC = jnp.mean(C * C, axis=1, keepdims=True)
        cross = jax.lax.dot_general(C, A, (((1,), (1,)), ((), ())),
                                    precision=jax.lax.Precision.HIGHEST,
                                    preferred_element_type=jnp.float32)
        var = vC + vA.reshape(1, N_BATCH) + cross * (2.0 / HID)
        inv = jax.lax.rsqrt(var + 1e-5)
        mu = mu_c + mu_a.reshape(1, N_BATCH)
        ab1 = a + b1
        t = c[:, None, :] + ab1[None, :, :]
        t = (t - mu[:, :, None]) * inv[:, :, None]
        t = t * g1_ref[...][None, :, :] + be1_ref[...][None, :, :]
        t = jnp.maximum(t, 0.01 * t)
        g = jnp.dot(t.reshape(KB * N_BATCH, HID), w2_ref[...],
                    preferred_element_type=jnp.float32)
        g = jax.nn.sigmoid(g + b2_ref[...])
        g3 = g.reshape(KB, N_BATCH, CD) * ps_ref[...].reshape(1, N_BATCH, 1)
        dmean = jnp.sum(g3, axis=1) * (1.0 / N_BATCH)
        tmp = 0.7 * cb + 0.3 * (cb + dmean)
        tmp_s[pl.ds(i * KB, KB), :] = tmp
        mix_s[pl.ds(i * KB, KB), :] = 0.7 * cb + 0.3 * tmp
        cbsq_s[pl.ds(i * KB, KB), :] = jnp.sum(cb * cb, axis=1, keepdims=True)
        tmpsq_s[pl.ds(i * KB, KB), :] = jnp.sum(tmp * tmp, axis=1,
                                                keepdims=True)

    @pl.when(i == NGATE)
    def _init():
        counts_acc[...] = jnp.zeros_like(counts_acc)
        loss_acc[0] = 0.0
        loss_acc[1] = 0.0

    @pl.when(i >= NGATE)
    def _main():
        xn = x_ref[0]
        x_sq = jnp.sum(xn * xn, axis=0, keepdims=True)
        xn2 = -2.0 * xn
        mm0 = jnp.dot(cb_ref[...], xn2,
                      preferred_element_type=jnp.float32)
        d0 = (x_sq + mm0) + cbsq_s[...]
        min0 = jnp.min(d0, axis=0, keepdims=True)
        eq0 = jnp.where(d0 == min0, 1.0, 0.0)
        ones_t = jnp.ones((T_LEN, 1), jnp.float32)
        counts_acc[...] += jnp.dot(eq0, ones_t,
                                   preferred_element_type=jnp.float32)
        loss_acc[0] += jnp.sum(min0)

        mm1 = jnp.dot(tmp_s[...], xn2,
                      preferred_element_type=jnp.float32)
        d1 = jnp.sqrt(jnp.maximum((x_sq + mm1) + tmpsq_s[...], 0.0))
        min1 = jnp.min(d1, axis=0, keepdims=True)
        iota = jax.lax.broadcasted_iota(
            jnp.int32, (NB, T_LEN), 0).astype(jnp.float32)
        idx1 = jnp.min(jnp.where(d1 == min1, iota, jnp.float32(NB)),
                       axis=0, keepdims=True)
        oh1 = jnp.where(iota == idx1, 1.0, 0.0)
        out = jax.lax.dot_general(mix_s[...], oh1, (((0,), (0,)), ((), ())),
                                  preferred_element_type=jnp.float32)
        xd_ref[0] = out
        diff = xn - out
        loss_acc[1] += jnp.sum(diff * diff)

    @pl.when(i == NGATE + N_BATCH - 1)
    def _final():
        cnt = counts_acc[...]
        p = cnt * (1.0 / jnp.sum(cnt))
        plogp = jnp.where(p > 0.0, p * jnp.log(jnp.maximum(p, 1e-30)), 0.0)
        perp_ref[...] = jnp.exp(-jnp.sum(plogp, axis=0, keepdims=True))
        denom = 1.0 / (jnp.float32(N_BATCH * T_LEN) * CD)
        commit = (0.7 * loss_acc[0] + 0.3 * loss_acc[1]) * denom
        commit_ref[...] = commit * jnp.ones((1, 1), jnp.float32)


def kernel(x, text_feat, part_sim, codebook, W1, b1, g1, be1, W2, b2):
    f32 = jnp.float32
    W1a = W1[:TXT]
    W1b = W1[TXT:]
    b1r = b1.reshape(1, HID)
    g1r = g1.reshape(1, HID)
    be1r = be1.reshape(1, HID)
    b2r = b2.reshape(1, CD)
    psr = part_sim.reshape(1, N_BATCH)

    full = lambda i: (0, 0)

    def x_map(i):
        return (jnp.maximum(i - NGATE, 0), 0, 0)

    x_d, commit, perp = pl.pallas_call(
        _fused_kernel,
        grid=(NGATE + N_BATCH,),
        in_specs=[
            pl.BlockSpec((N_BATCH, TXT), full),
            pl.BlockSpec((1, N_BATCH), full),
            pl.BlockSpec((NB, CD), full),
            pl.BlockSpec((TXT, HID), full),
            pl.BlockSpec((CD, HID), full),
            pl.BlockSpec((1, HID), full),
            pl.BlockSpec((1, HID), full),
            pl.BlockSpec((1, HID), full),
            pl.BlockSpec((HID, CD), full),
            pl.BlockSpec((1, CD), full),
            pl.BlockSpec((1, CD, T_LEN), x_map),
        ],
        out_specs=[
            pl.BlockSpec((1, CD, T_LEN), x_map),
            pl.BlockSpec((1, 1), full),
            pl.BlockSpec((1, 1), full),
        ],
        out_shape=[
            jax.ShapeDtypeStruct((N_BATCH, CD, T_LEN), f32),
            jax.ShapeDtypeStruct((1, 1), f32),
            jax.ShapeDtypeStruct((1, 1), f32),
        ],
        scratch_shapes=[
            pltpu.VMEM((NB, CD), f32),
            pltpu.VMEM((NB, CD), f32),
            pltpu.VMEM((NB, 1), f32),
            pltpu.VMEM((NB, 1), f32),
            pltpu.VMEM((NB, 1), f32),
            pltpu.SMEM((2,), f32),
        ],
    )(text_feat, psr, codebook, W1a, W1b, b1r, g1r, be1r, W2, b2r, x)

    return (x_d, commit[0, 0], perp[0, 0])
